# baseline (speedup 1.0000x reference)
import functools

import jax
import jax.numpy as jnp
from jax import lax
from jax.experimental import pallas as pl
from jax.experimental.pallas import tpu as pltpu
from jax.experimental.pallas import tpu_sc as plsc

N0 = 100000
N1 = 10000
N2 = 2048
D = 128
NC = 2
NS = 16
NW = NC * NS
STEP = 128
L = 16
NB = 2

NP1 = 10240
NP2 = 2048


def _seg_sums_sc(table, src, dst, zeros2d, zeros1d, np_rows):
    E = src.shape[0] - NB * STEP
    ept = E // NW
    nsteps = ept // STEP
    nhalf = nsteps // (2 * NB)
    rpz = np_rows // NS
    mesh = plsc.VectorSubcoreMesh(core_axis_name="c", subcore_axis_name="s")

    @functools.partial(
        pl.kernel,
        out_type=(
            jax.ShapeDtypeStruct((NC, np_rows, D), jnp.float32),
            jax.ShapeDtypeStruct((NC * np_rows,), jnp.float32),
        ),
        mesh=mesh,
        scratch_types=(
            [pltpu.VMEM((STEP,), jnp.int32) for _ in range(4 * NB)]
            + [
                pltpu.VMEM((NB, STEP, D), jnp.float32),
                pltpu.VMEM((STEP,), jnp.float32),
                pltpu.VMEM_SHARED((np_rows, D), jnp.float32),
                pltpu.VMEM_SHARED((np_rows,), jnp.float32),
            ]
            + [pltpu.SemaphoreType.DMA((NB,)) for _ in range(5)]
        ),
    )
    def k(table_h, src_h, dst_h, z2_h, z1_h, sums_h, cnt_h, *scr):
        sbuf = scr[:2 * NB]
        dbuf = scr[2 * NB:4 * NB]
        rows_v, ones_v, acc_sh, cnt_sh = scr[4 * NB:4 * NB + 4]
        gsem, ssem, csem, isems, isemd = scr[4 * NB + 4:]
        cid = lax.axis_index("c")
        sid = lax.axis_index("s")
        wid = sid * NC + cid
        r0 = sid * rpz
        base = wid * ept

        pltpu.sync_copy(z2_h.at[pl.ds(0, rpz)], acc_sh.at[pl.ds(r0, rpz)])
        pltpu.sync_copy(z1_h.at[pl.ds(0, rpz)], cnt_sh.at[pl.ds(r0, rpz)])
        ov = jnp.ones((L,), jnp.float32)
        for q in range(STEP // L):
            ones_v[pl.ds(q * L, L)] = ov
        plsc.subcore_barrier()

        def idx_load(B, step):
            off = pl.multiple_of(base + step * STEP, STEP)
            pltpu.async_copy(src_h.at[pl.ds(off, STEP)], sbuf[B],
                             isems.at[B % NB])
            pltpu.async_copy(dst_h.at[pl.ds(off, STEP)], dbuf[B],
                             isemd.at[B % NB])

        def idx_wait(B):
            pltpu.make_async_copy(src_h.at[pl.ds(0, STEP)], sbuf[B],
                                  isems.at[B % NB]).wait()
            pltpu.make_async_copy(dst_h.at[pl.ds(0, STEP)], dbuf[B],
                                  isemd.at[B % NB]).wait()

        def scat_wait(b, prev_parity):
            pltpu.make_async_copy(rows_v.at[b],
                                  acc_sh.at[dbuf[prev_parity * NB + b]],
                                  ssem.at[b]).wait()
            pltpu.make_async_copy(ones_v,
                                  cnt_sh.at[dbuf[prev_parity * NB + b]],
                                  csem.at[b]).wait()

        def half(i2, p, first):
            j0 = (2 * i2 + p) * NB
            gd = []
            for b in range(NB):
                if first is not None:
                    @pl.when(first)
                    def _():
                        scat_wait(b, 1 - p)
                else:
                    scat_wait(b, 1 - p)
                idx_wait(p * NB + b)
                gd.append([pltpu.async_copy(
                    table_h.at[sbuf[p * NB + b].at[pl.ds(hh, 64)]],
                    rows_v.at[b, pl.ds(hh, 64)], gsem.at[b])
                    for hh in (0, 64)])
            for b in range(NB):
                for g in gd[b]:
                    g.wait()
                pltpu.async_copy(rows_v.at[b], acc_sh.at[dbuf[p * NB + b]],
                                 ssem.at[b], add=True)
                pltpu.async_copy(ones_v, cnt_sh.at[dbuf[p * NB + b]],
                                 csem.at[b], add=True)
                idx_load((1 - p) * NB + b, j0 + NB + b)

        for b in range(NB):
            idx_load(b, b)

        @pl.loop(0, nhalf)
        def _(i2):
            half(i2, 0, i2 > 0)
            half(i2, 1, None)

        for b in range(NB):
            scat_wait(b, 1)
            idx_wait(b)

        plsc.subcore_barrier()
        pltpu.sync_copy(acc_sh.at[pl.ds(r0, rpz)],
                        sums_h.at[cid, pl.ds(r0, rpz)])
        pltpu.sync_copy(cnt_sh.at[pl.ds(r0, rpz)],
                        cnt_h.at[pl.ds(cid * np_rows + r0, rpz)])

    sums, cnt_flat = k(table, src, dst, zeros2d, zeros1d)
    return sums, cnt_flat.reshape(NC, np_rows)


def _sage_linear_tc(sums, cnts, xsrc, wl_t, wr_t, bias, nrows, blk, relu):
    np_rows = sums.shape[1]

    def body(sums_ref, cnt_ref, x_ref, wl_ref, wr_ref, b_ref, o_ref):
        i = pl.program_id(0)
        s = sums_ref[0] + sums_ref[1]
        c = cnt_ref[0, pl.ds(i * blk, blk)] + cnt_ref[1, pl.ds(i * blk, blk)]
        inv = 1.0 / jnp.maximum(c, 1.0)
        agg = s * inv[:, None]
        r = (jnp.dot(agg, wl_ref[...], preferred_element_type=jnp.float32)
             + jnp.dot(x_ref[...], wr_ref[...], preferred_element_type=jnp.float32)
             + b_ref[...])
        if relu:
            r = jnp.maximum(r, 0.0)
        o_ref[...] = r

    return pl.pallas_call(
        body,
        grid=(nrows // blk,),
        in_specs=[
            pl.BlockSpec((NC, blk, D), lambda i: (0, i, 0)),
            pl.BlockSpec((NC, np_rows), lambda i: (0, 0)),
            pl.BlockSpec((blk, D), lambda i: (i, 0)),
            pl.BlockSpec((D, D), lambda i: (0, 0)),
            pl.BlockSpec((D, D), lambda i: (0, 0)),
            pl.BlockSpec((1, D), lambda i: (0, 0)),
        ],
        out_specs=pl.BlockSpec((blk, D), lambda i: (i, 0)),
        out_shape=jax.ShapeDtypeStruct((nrows, D), jnp.float32),
    )(sums, cnts, xsrc, wl_t, wr_t, bias)


def _pad_edges(src, dst, scrap):
    e = src.shape[0]
    chunk = NW * STEP * 2 * NB
    ep = -(-e // chunk) * chunk + NB * STEP
    src = jnp.concatenate([src, jnp.zeros((ep - e,), jnp.int32)])
    dst = jnp.concatenate([dst, jnp.full((ep - e,), scrap, jnp.int32)])
    return src, dst


def kernel(x, src0, dst0, src1, dst1, n1, n2, Wl0, bl0, Wr0, Wl1, bl1, Wr1):
    src0, dst0 = _pad_edges(src0, dst0, N1)
    src1, dst1 = _pad_edges(src1, dst1, N2 - 1)

    zeros2d = jnp.zeros((NP1 // NS, D), jnp.float32)
    zeros1d = jnp.zeros((NP1 // NS,), jnp.float32)
    zero = (jnp.asarray(n1, jnp.int32) - N1
            + jnp.asarray(n2, jnp.int32) - N2).astype(jnp.float32)

    sums0, cnt0 = _seg_sums_sc(x, src0, dst0, zeros2d, zeros1d, NP1)
    h = _sage_linear_tc(sums0, cnt0, x, Wl0.T, Wr0.T, bl0[None, :],
                        NP1, 1024, relu=True)

    sums1, cnt1 = _seg_sums_sc(h, src1, dst1, zeros2d, zeros1d, NP2)
    out = _sage_linear_tc(sums1, cnt1, h, Wl1.T, Wr1.T, (bl1 + zero)[None, :],
                          N2, 1024, relu=False)
    return out
